# baseline (speedup 1.0000x reference)
import jax
import jax.numpy as jnp
from jax.experimental import pallas as pl

_B, _C, _F, _N = 4, 128, 12, 512


def _body(y_ref, x_ref, we_ref, wl_ref, b_ref, o_ref):
    for f in range(_F):
        a = y_ref[0, f] * we_ref[...]
        m = jnp.dot(x_ref[0, :, f * _N:(f + 1) * _N], a,
                    preferred_element_type=jnp.float32)
        h = jnp.dot(wl_ref[...], m,
                    preferred_element_type=jnp.float32) + b_ref[...]
        o_ref[0, :, f * _N:(f + 1) * _N] = jnp.maximum(h, 0.0)


@jax.jit
def kernel(Y, infos, W_edge, W_lin, b_lin):
    wl = W_lin * jnp.float32(1.0 / _N)
    b2 = b_lin.reshape(_C, 1)
    out = pl.pallas_call(
        _body,
        grid=(_B,),
        in_specs=[
            pl.BlockSpec((1, _F, _N, _N), lambda b: (b, 0, 0, 0)),
            pl.BlockSpec((1, _C, _F * _N), lambda b: (b, 0, 0)),
            pl.BlockSpec((_N, _N), lambda b: (0, 0)),
            pl.BlockSpec((_C, _C), lambda b: (0, 0)),
            pl.BlockSpec((_C, 1), lambda b: (0, 0)),
        ],
        out_specs=pl.BlockSpec((1, _C, _F * _N), lambda b: (b, 0, 0)),
        out_shape=jax.ShapeDtypeStruct((_B, _C, _F * _N), jnp.float32),
    )(Y, infos.reshape(_B, _C, _F * _N), W_edge, wl, b2)
    return out.reshape(_B, _C, _F, _N)


# speedup vs baseline: 3.0764x; 3.0764x over previous
import jax
import jax.numpy as jnp
from jax.experimental import pallas as pl

_B, _C, _F, _N = 4, 128, 12, 512
_G = 6


def _body(y_ref, x_ref, we_ref, wl_ref, b_ref, o_ref):
    for g in range(_G):
        a = y_ref[0, g] * we_ref[...]
        m = jnp.dot(x_ref[0, g], a,
                    preferred_element_type=jnp.float32)
        h = jnp.dot(wl_ref[...], m,
                    preferred_element_type=jnp.float32) + b_ref[...]
        o_ref[0, g] = jnp.maximum(h, 0.0)


@jax.jit
def kernel(Y, infos, W_edge, W_lin, b_lin):
    wl = W_lin * jnp.float32(1.0 / _N)
    b2 = b_lin.reshape(_C, 1)
    out = pl.pallas_call(
        _body,
        grid=(_B, _F // _G),
        in_specs=[
            pl.BlockSpec((1, _G, _N, _N), lambda b, f: (b, f, 0, 0)),
            pl.BlockSpec((1, _G, _C, _N), lambda b, f: (b, f, 0, 0)),
            pl.BlockSpec((_N, _N), lambda b, f: (0, 0)),
            pl.BlockSpec((_C, _C), lambda b, f: (0, 0)),
            pl.BlockSpec((_C, 1), lambda b, f: (0, 0)),
        ],
        out_specs=pl.BlockSpec((1, _G, _C, _N), lambda b, f: (b, f, 0, 0)),
        out_shape=jax.ShapeDtypeStruct((_B, _F, _C, _N), jnp.float32),
    )(Y, jnp.transpose(infos, (0, 2, 1, 3)), W_edge, wl, b2)
    return jnp.transpose(out, (0, 2, 1, 3))
